# baseline (speedup 1.0000x reference)
import functools

import jax
import jax.numpy as jnp
from jax import lax
from jax.experimental import pallas as pl
from jax.experimental.pallas import tpu as pltpu
from jax.experimental.pallas import tpu_sc as plsc

N = 10000
E = 320000
D = 128
EPS = 1e-5

NP = 10240
CHUNK = 128
NUM_WORKERS = 32
EPT = 10240
EPAD = EPT * NUM_WORKERS
NCHUNK = EPT // CHUNK
ROWS_PT = NP // 16

_mesh = plsc.VectorSubcoreMesh(core_axis_name="c", subcore_axis_name="s")


@functools.partial(
    pl.kernel,
    out_type=jax.ShapeDtypeStruct((2, NP, D), jnp.float32),
    mesh=_mesh,
    scratch_types=[
        pltpu.VMEM((CHUNK,), jnp.int32),
        pltpu.VMEM((CHUNK, D), jnp.float32),
        pltpu.VMEM((16, D), jnp.float32),
        pltpu.VMEM_SHARED((NP, D), jnp.float32),
    ],
)
def _deg_kernel(dst_hbm, out_hbm, didx, ones_v, zero_v, acc):
    c = lax.axis_index("c")
    s = lax.axis_index("s")
    wid = s * 2 + c

    for i in range(16):
        for j in range(D // 16):
            zero_v[i, pl.ds(j * 16, 16)] = jnp.zeros((16,), jnp.float32)
    for i in range(CHUNK):
        for j in range(D // 16):
            ones_v[i, pl.ds(j * 16, 16)] = jnp.ones((16,), jnp.float32)

    for k in range(ROWS_PT // 16):
        pltpu.sync_copy(zero_v, acc.at[pl.ds(s * ROWS_PT + k * 16, 16)])
    plsc.subcore_barrier()

    def body(g, carry):
        base = wid * EPT + g * CHUNK
        pltpu.sync_copy(dst_hbm.at[pl.ds(base, CHUNK)], didx)
        pltpu.sync_copy(ones_v, acc.at[didx], add=True)
        return carry

    lax.fori_loop(0, NCHUNK, body, 0)
    plsc.subcore_barrier()

    pltpu.sync_copy(
        acc.at[pl.ds(s * ROWS_PT, ROWS_PT)],
        out_hbm.at[c, pl.ds(s * ROWS_PT, ROWS_PT)],
    )


@functools.partial(
    pl.kernel,
    out_type=jax.ShapeDtypeStruct((2, NP, D), jnp.float32),
    mesh=_mesh,
    scratch_types=[
        pltpu.VMEM((CHUNK,), jnp.int32),
        pltpu.VMEM((CHUNK,), jnp.int32),
        pltpu.VMEM((CHUNK, D), jnp.float32),
        pltpu.VMEM((16, D), jnp.float32),
        pltpu.VMEM_SHARED((NP, D), jnp.float32),
        pltpu.SemaphoreType.DMA,
    ],
)
def _agg_kernel(ht_hbm, src_hbm, dst_hbm, out_hbm, sidx, didx, rows, zero_v, acc, sem):
    c = lax.axis_index("c")
    s = lax.axis_index("s")
    wid = s * 2 + c

    for i in range(16):
        for j in range(D // 16):
            zero_v[i, pl.ds(j * 16, 16)] = jnp.zeros((16,), jnp.float32)

    for k in range(ROWS_PT // 16):
        pltpu.sync_copy(zero_v, acc.at[pl.ds(s * ROWS_PT + k * 16, 16)])
    plsc.subcore_barrier()

    def body(g, carry):
        base = wid * EPT + g * CHUNK
        pltpu.sync_copy(src_hbm.at[pl.ds(base, CHUNK)], sidx)
        pltpu.sync_copy(dst_hbm.at[pl.ds(base, CHUNK)], didx)
        pltpu.async_copy(ht_hbm.at[sidx], rows, sem).wait()
        pltpu.sync_copy(rows, acc.at[didx], add=True)
        return carry

    lax.fori_loop(0, NCHUNK, body, 0)
    plsc.subcore_barrier()

    pltpu.sync_copy(
        acc.at[pl.ds(s * ROWS_PT, ROWS_PT)],
        out_hbm.at[c, pl.ds(s * ROWS_PT, ROWS_PT)],
    )


def _t1_body(xp_ref, w1_ref, cnt_ref, ht_ref, dinv_ref):
    cnt = cnt_ref[0, :, 0:1] + cnt_ref[1, :, 0:1]
    deg = cnt + 1.0
    row = lax.broadcasted_iota(jnp.int32, (NP, 1), 0)
    dinv = jnp.where(row < N, lax.rsqrt(deg), 0.0)
    ht = jnp.dot(xp_ref[...], w1_ref[...], preferred_element_type=jnp.float32)
    ht_ref[...] = ht * dinv
    dinv_ref[...] = dinv


def _t2_body(p_ref, ht1_ref, dinv_ref, b1_ref, g_ref, be_ref, w2_ref, ht2_ref):
    agg = p_ref[0] + p_ref[1] + ht1_ref[...]
    o1 = agg * dinv_ref[...] + b1_ref[...]
    o1r = o1[:N]
    mean = jnp.mean(o1r, axis=0, keepdims=True)
    var = jnp.mean((o1r - mean) ** 2, axis=0, keepdims=True)
    h2 = jnp.maximum((o1 - mean) * lax.rsqrt(var + EPS) * g_ref[...] + be_ref[...], 0.0)
    ht2 = jnp.dot(h2, w2_ref[...], preferred_element_type=jnp.float32)
    ht2_ref[...] = ht2 * dinv_ref[...]


def _t3_body(p_ref, ht2_ref, dinv_ref, b2_ref, out_ref):
    agg = p_ref[0, :N] + p_ref[1, :N] + ht2_ref[:N]
    out_ref[...] = agg * dinv_ref[:N] + b2_ref[...]


_t1 = pl.pallas_call(
    _t1_body,
    out_shape=(
        jax.ShapeDtypeStruct((NP, D), jnp.float32),
        jax.ShapeDtypeStruct((NP, 1), jnp.float32),
    ),
)

_t2 = pl.pallas_call(
    _t2_body,
    out_shape=jax.ShapeDtypeStruct((NP, D), jnp.float32),
)

_t3 = pl.pallas_call(
    _t3_body,
    out_shape=jax.ShapeDtypeStruct((N, D), jnp.float32),
)


def kernel(x, edge_index, W1, b1, gamma, beta, W2, b2):
    src = edge_index[0]
    dst = edge_index[1]
    pad = jnp.full((EPAD - E,), N, dtype=jnp.int32)
    srcp = jnp.concatenate([src, pad])
    dstp = jnp.concatenate([dst, pad])
    xp = jnp.pad(x, ((0, NP - N), (0, 0)))

    cnt = _deg_kernel(dstp)
    ht1, dinv = _t1(xp, W1, cnt)
    p1 = _agg_kernel(ht1, srcp, dstp)
    ht2 = _t2(p1, ht1, dinv, b1.reshape(1, D), gamma.reshape(1, D),
              beta.reshape(1, D), W2)
    p2 = _agg_kernel(ht2, srcp, dstp)
    out = _t3(p2, ht2, dinv, b2.reshape(1, D))
    return out


# speedup vs baseline: 1.1516x; 1.1516x over previous
import functools

import jax
import jax.numpy as jnp
from jax import lax
from jax.experimental import pallas as pl
from jax.experimental.pallas import tpu as pltpu
from jax.experimental.pallas import tpu_sc as plsc

N = 10000
E = 320000
D = 128
EPS = 1e-5

NP = 10240
CHUNK = 128
NUM_WORKERS = 32
EPT = 10240
EPAD = EPT * NUM_WORKERS
NCHUNK = EPT // CHUNK
IDXBLK = 8
ROWS_PT = NP // 16

_mesh = plsc.VectorSubcoreMesh(core_axis_name="c", subcore_axis_name="s")


@functools.partial(
    pl.kernel,
    out_type=jax.ShapeDtypeStruct((2, NP, D), jnp.float32),
    mesh=_mesh,
    scratch_types=[
        pltpu.VMEM((NCHUNK, CHUNK), jnp.int32),
        pltpu.VMEM((CHUNK, D), jnp.float32),
        pltpu.VMEM((16, D), jnp.float32),
        pltpu.VMEM_SHARED((NP, D), jnp.float32),
    ],
)
def _deg_kernel(dst_hbm, out_hbm, didx_all, ones_v, zero_v, acc):
    c = lax.axis_index("c")
    s = lax.axis_index("s")
    wid = s * 2 + c

    pltpu.sync_copy(dst_hbm.at[pl.ds(wid * NCHUNK, NCHUNK)], didx_all)

    for i in range(16):
        for j in range(D // 16):
            zero_v[i, pl.ds(j * 16, 16)] = jnp.zeros((16,), jnp.float32)
    for i in range(CHUNK):
        for j in range(D // 16):
            ones_v[i, pl.ds(j * 16, 16)] = jnp.ones((16,), jnp.float32)

    for k in range(ROWS_PT // 16):
        pltpu.sync_copy(zero_v, acc.at[pl.ds(s * ROWS_PT + k * 16, 16)])
    plsc.subcore_barrier()

    def body(g, carry):
        pltpu.sync_copy(ones_v, acc.at[didx_all.at[g]], add=True)
        return carry

    lax.fori_loop(0, NCHUNK, body, 0)
    plsc.subcore_barrier()

    pltpu.sync_copy(
        acc.at[pl.ds(s * ROWS_PT, ROWS_PT)],
        out_hbm.at[c, pl.ds(s * ROWS_PT, ROWS_PT)],
    )


@functools.partial(
    pl.kernel,
    out_type=jax.ShapeDtypeStruct((2, NP, D), jnp.float32),
    mesh=_mesh,
    scratch_types=[
        pltpu.VMEM((IDXBLK, CHUNK), jnp.int32),
        pltpu.VMEM((IDXBLK, CHUNK), jnp.int32),
        pltpu.VMEM((CHUNK, D), jnp.float32),
        pltpu.VMEM((CHUNK, D), jnp.float32),
        pltpu.VMEM((16, D), jnp.float32),
        pltpu.VMEM_SHARED((NP, D), jnp.float32),
        pltpu.SemaphoreType.DMA,
        pltpu.SemaphoreType.DMA,
    ],
)
def _agg_kernel(ht_hbm, src_hbm, dst_hbm, out_hbm, sidx_blk, didx_blk,
                rows0, rows1, zero_v, acc, sem0, sem1):
    c = lax.axis_index("c")
    s = lax.axis_index("s")
    wid = s * 2 + c

    for i in range(16):
        for j in range(D // 16):
            zero_v[i, pl.ds(j * 16, 16)] = jnp.zeros((16,), jnp.float32)

    for k in range(ROWS_PT // 16):
        pltpu.sync_copy(zero_v, acc.at[pl.ds(s * ROWS_PT + k * 16, 16)])
    plsc.subcore_barrier()

    def body(b, carry):
        base = wid * NCHUNK + b * IDXBLK
        pltpu.sync_copy(src_hbm.at[pl.ds(base, IDXBLK)], sidx_blk)
        pltpu.sync_copy(dst_hbm.at[pl.ds(base, IDXBLK)], didx_blk)
        pltpu.async_copy(ht_hbm.at[sidx_blk.at[0]], rows0, sem0)
        for j in range(IDXBLK):
            rows, sem = (rows0, sem0) if j % 2 == 0 else (rows1, sem1)
            nrows, nsem = (rows1, sem1) if j % 2 == 0 else (rows0, sem0)
            pltpu.make_async_copy(ht_hbm.at[sidx_blk.at[j]], rows, sem).wait()
            if j + 1 < IDXBLK:
                pltpu.async_copy(ht_hbm.at[sidx_blk.at[j + 1]], nrows, nsem)
            pltpu.sync_copy(rows, acc.at[didx_blk.at[j]], add=True)
        return carry

    lax.fori_loop(0, NCHUNK // IDXBLK, body, 0)
    plsc.subcore_barrier()

    pltpu.sync_copy(
        acc.at[pl.ds(s * ROWS_PT, ROWS_PT)],
        out_hbm.at[c, pl.ds(s * ROWS_PT, ROWS_PT)],
    )


def _t1_body(xp_ref, w1_ref, cnt_ref, ht_ref, dinv_ref):
    cnt = cnt_ref[0, :, 0:1] + cnt_ref[1, :, 0:1]
    deg = cnt + 1.0
    row = lax.broadcasted_iota(jnp.int32, (NP, 1), 0)
    dinv = jnp.where(row < N, lax.rsqrt(deg), 0.0)
    ht = jnp.dot(xp_ref[...], w1_ref[...], preferred_element_type=jnp.float32)
    ht_ref[...] = ht * dinv
    dinv_ref[...] = dinv


def _t2_body(p_ref, ht1_ref, dinv_ref, b1_ref, g_ref, be_ref, w2_ref, ht2_ref):
    agg = p_ref[0] + p_ref[1] + ht1_ref[...]
    o1 = agg * dinv_ref[...] + b1_ref[...]
    o1r = o1[:N]
    mean = jnp.mean(o1r, axis=0, keepdims=True)
    var = jnp.mean((o1r - mean) ** 2, axis=0, keepdims=True)
    h2 = jnp.maximum((o1 - mean) * lax.rsqrt(var + EPS) * g_ref[...] + be_ref[...], 0.0)
    ht2 = jnp.dot(h2, w2_ref[...], preferred_element_type=jnp.float32)
    ht2_ref[...] = ht2 * dinv_ref[...]


def _t3_body(p_ref, ht2_ref, dinv_ref, b2_ref, out_ref):
    agg = p_ref[0, :N] + p_ref[1, :N] + ht2_ref[:N]
    out_ref[...] = agg * dinv_ref[:N] + b2_ref[...]


_t1 = pl.pallas_call(
    _t1_body,
    out_shape=(
        jax.ShapeDtypeStruct((NP, D), jnp.float32),
        jax.ShapeDtypeStruct((NP, 1), jnp.float32),
    ),
)

_t2 = pl.pallas_call(
    _t2_body,
    out_shape=jax.ShapeDtypeStruct((NP, D), jnp.float32),
)

_t3 = pl.pallas_call(
    _t3_body,
    out_shape=jax.ShapeDtypeStruct((N, D), jnp.float32),
)


def kernel(x, edge_index, W1, b1, gamma, beta, W2, b2):
    src = edge_index[0]
    dst = edge_index[1]
    pad = jnp.full((EPAD - E,), N, dtype=jnp.int32)
    srcp = jnp.concatenate([src, pad]).reshape(EPAD // CHUNK, CHUNK)
    dstp = jnp.concatenate([dst, pad]).reshape(EPAD // CHUNK, CHUNK)
    xp = jnp.pad(x, ((0, NP - N), (0, 0)))

    cnt = _deg_kernel(dstp)
    ht1, dinv = _t1(xp, W1, cnt)
    p1 = _agg_kernel(ht1, srcp, dstp)
    ht2 = _t2(p1, ht1, dinv, b1.reshape(1, D), gamma.reshape(1, D),
              beta.reshape(1, D), W2)
    p2 = _agg_kernel(ht2, srcp, dstp)
    out = _t3(p2, ht2, dinv, b2.reshape(1, D))
    return out
